# baseline (speedup 1.0000x reference)
import numpy as np
import jax
import jax.numpy as jnp
from jax import lax
from jax.experimental import pallas as pl

N = 10000
DN = (((1,), (0,)), ((), ()))
R_INV = np.float32(1e-4)


def _chunk_body(bm, n, c):
    nfull, rem = N // c, N % c

    def body(a_ref, b_ref, o_ref):
        def dot_chunk(kc, width):
            a = a_ref[:, pl.ds(kc, width)].astype(jnp.bfloat16)
            b = b_ref[pl.ds(kc, width), :].astype(jnp.bfloat16)
            return lax.dot_general(a, b, DN,
                                   preferred_element_type=jnp.float32)

        def step(i, acc):
            return acc + dot_chunk(i * c, c)

        acc = lax.fori_loop(0, nfull, step,
                            jnp.zeros((bm, n), jnp.float32))
        if rem:
            acc = acc + dot_chunk(nfull * c, rem)
        o_ref[...] = acc
    return body


def _pallas_mm_chunk(a, b, bm=400, c=256):
    m, k = a.shape
    _, n = b.shape
    return pl.pallas_call(
        _chunk_body(bm, n, c),
        grid=(m // bm,),
        in_specs=[pl.BlockSpec((bm, k), lambda i: (i, 0)),
                  pl.BlockSpec((k, n), lambda i: (0, 0))],
        out_specs=pl.BlockSpec((bm, n), lambda i: (i, 0)),
        out_shape=jax.ShapeDtypeStruct((m, n), jnp.float32),
    )(a, b)


def _sfold(acc):
    b1 = acc[0:4, :] + acc[4:8, :]
    b2 = b1[0:2, :] + b1[2:4, :]
    return b2[0:1, :] + b2[1:2, :]


def _redC_body(c):
    def body(x_ref, o_ref):
        def step(v, acc):
            return acc + x_ref[pl.ds(128 * v, 128), :]

        acc = lax.fori_loop(0, 78, step, jnp.zeros((128, c), jnp.float32))
        tail = jnp.concatenate(
            [x_ref[pl.ds(9984, 16), :],
             jnp.zeros((112, c), jnp.float32)], axis=0)
        acc = acc + tail
        acc2 = acc[0:8, :]
        for q in range(1, 16):
            acc2 = acc2 + acc[8 * q:8 * q + 8, :]
        o_ref[...] = _sfold(acc2)
    return body


def _reduceC(arr):
    c = arr.shape[1]
    return pl.pallas_call(
        _redC_body(c),
        out_shape=jax.ShapeDtypeStruct((1, c), jnp.float32),
    )(arr)


def _small_mm_body(a_ref, b_ref, o_ref):
    a = a_ref[...].astype(jnp.bfloat16)
    b = b_ref[...].astype(jnp.bfloat16)
    o_ref[...] = lax.dot_general(a, b, DN, preferred_element_type=jnp.float32)


def _pallas_mm_small(a, b):
    m, k = a.shape
    _, n = b.shape
    return pl.pallas_call(
        _small_mm_body,
        out_shape=jax.ShapeDtypeStruct((m, n), jnp.float32),
    )(a, b)


def kernel(x, adj, W_self_0, W_neigh_0, b_0, gamma_0, beta_0,
           W_self_1, W_neigh_1, b_1, gamma_1, beta_1,
           W_self_2, W_neigh_2, b_2, gamma_2, beta_2):
    params = [
        (W_self_0, W_neigh_0, b_0, gamma_0, beta_0),
        (W_self_1, W_neigh_1, b_1, gamma_1, beta_1),
        (W_self_2, W_neigh_2, b_2, gamma_2, beta_2),
    ]
    deg = jnp.maximum(adj.sum(axis=1, keepdims=True), 1.0)
    h = x
    for (Ws, Wn, b, g, bt) in params:
        neigh = _pallas_mm_chunk(adj, h) / deg
        pre = jax.nn.relu(_pallas_mm_small(h, Ws)
                          + _pallas_mm_small(neigh, Wn) + b)
        m = _reduceC(pre) * R_INV
        centered = pre - m
        v = _reduceC(centered * centered) * R_INV
        s = jnp.sqrt(v + 1e-5)
        h = centered / s * g + bt
    return _reduceC(h) * R_INV
